# baseline (speedup 1.0000x reference)
import functools

import jax
import jax.numpy as jnp
from jax import lax
from jax.experimental import pallas as pl
from jax.experimental.pallas import tpu as pltpu
from jax.experimental.pallas import tpu_sc as plsc

_SC_ROWS = 256
_LANES = 16


def _prep_body(lw_ref, keys_ref, t_ref, dlo_ref, base_ref):
    blk = lw_ref[...]
    L = blk.shape[0]
    row0 = blk[0:1, :]
    eq = (blk == row0).astype(jnp.float32)
    t_ref[...] = jnp.sum(eq, axis=0, keepdims=True)
    keys = keys_ref[...]
    hi = blk[L - 1:L, :]
    dlo_ref[...] = keys * (row0 - hi)
    base_ref[...] = jnp.sum(keys * hi, axis=0, keepdims=True)


def _tc_body(x_ref, t_ref, dlo_ref, base_ref, out_ref, *, L):
    x = x_ref[...]
    BB, C = x.shape
    DB = out_ref.shape[1]
    idx = jnp.clip(jnp.round(x * (L - 1)), 0.0, L - 1.0)
    t = t_ref[...]
    dlo = dlo_ref[...]
    acc = jnp.broadcast_to(base_ref[...], (BB, DB))
    for c in range(C):
        idx_c = jnp.broadcast_to(idx[:, c:c + 1], (BB, DB))
        mask = idx_c < t
        acc = acc + jnp.where(mask, jnp.broadcast_to(dlo[c:c + 1, :], (BB, DB)), 0.0)
    out_ref[...] = jnp.where(acc > 0, 1.0, -1.0)


def _make_sc_kernel(n_rows, C, D, L):
    info = plsc.get_sparse_core_info()
    nw = info.num_cores * info.num_subcores
    assert n_rows % nw == 0
    bpw = n_rows // nw
    nk = D // _LANES
    CP = 32
    ni = (bpw * CP) // _LANES
    mesh = plsc.VectorSubcoreMesh(core_axis_name="c", subcore_axis_name="s")

    @functools.partial(
        pl.kernel,
        mesh=mesh,
        out_type=jax.ShapeDtypeStruct((n_rows * D,), jnp.float32),
        scratch_types=[
            pltpu.VMEM((bpw * CP,), jnp.float32),
            pltpu.VMEM((bpw * CP,), jnp.float32),
            pltpu.VMEM((C * D,), jnp.float32),
            pltpu.VMEM((D,), jnp.float32),
            pltpu.VMEM((D,), jnp.float32),
            pltpu.VMEM((D,), jnp.float32),
        ],
    )
    def sc_kernel(x_hbm, dlo_hbm, t_hbm, base_hbm, out_hbm,
                  x_v, idx_v, dlo_v, t_v, base_v, row_v):
        wid = lax.axis_index("s") * info.num_cores + lax.axis_index("c")
        b0 = wid * bpw
        pltpu.sync_copy(x_hbm.at[pl.ds(b0 * CP, bpw * CP)], x_v)
        pltpu.sync_copy(dlo_hbm, dlo_v)
        pltpu.sync_copy(t_hbm, t_v)
        pltpu.sync_copy(base_hbm, base_v)

        def quant(i, _):
            xv = x_v[pl.ds(i * _LANES, _LANES)]
            y = xv * float(L - 1) + 0.5
            iv = y.astype(jnp.int32)
            fv = iv.astype(jnp.float32)
            odd = jnp.bitwise_and(iv, 1).astype(jnp.float32)
            r = fv - jnp.where(fv == y, odd, 0.0)
            r = jnp.minimum(jnp.maximum(r, 0.0), float(L - 1))
            idx_v[pl.ds(i * _LANES, _LANES)] = r
            return _
        lax.fori_loop(0, ni, quant, None)

        def body_b(b, _):
            v0 = idx_v[pl.ds(b * CP, _LANES)]
            v1 = idx_v[pl.ds(b * CP + _LANES, _LANES)]
            dnums = lax.GatherDimensionNumbers(
                offset_dims=(), collapsed_slice_dims=(0,), start_index_map=(0,))
            splats = [
                lax.gather(v0 if c < _LANES else v1,
                           jnp.full((_LANES, 1), c % _LANES, jnp.int32),
                           dnums, slice_sizes=(1,),
                           mode=lax.GatherScatterMode.PROMISE_IN_BOUNDS)
                for c in range(C)
            ]

            def body_k(k, _):
                sl = pl.ds(k * _LANES, _LANES)
                t_c = t_v[sl]
                acc = base_v[sl]
                for c in range(C):
                    m = splats[c] < t_c
                    acc = acc + jnp.where(m, dlo_v[pl.ds(c * D + k * _LANES, _LANES)], 0.0)
                row_v[sl] = jnp.where(acc > 0.0, 1.0, -1.0)
                return _
            lax.fori_loop(0, nk, body_k, None)
            pltpu.sync_copy(row_v, out_hbm.at[pl.ds((b0 + b) * D, D)])
            return _
        lax.fori_loop(0, bpw, body_b, None)

    return sc_kernel


@jax.jit
def kernel(input, keys_weight, level_weight):
    B, C = input.shape
    L, D = level_weight.shape
    t, dlo, base = pl.pallas_call(
        _prep_body,
        grid=(1,),
        in_specs=[
            pl.BlockSpec((L, D), lambda i: (0, 0)),
            pl.BlockSpec((C, D), lambda i: (0, 0)),
        ],
        out_specs=[
            pl.BlockSpec((1, D), lambda i: (0, 0)),
            pl.BlockSpec((C, D), lambda i: (0, 0)),
            pl.BlockSpec((1, D), lambda i: (0, 0)),
        ],
        out_shape=[
            jax.ShapeDtypeStruct((1, D), jnp.float32),
            jax.ShapeDtypeStruct((C, D), jnp.float32),
            jax.ShapeDtypeStruct((1, D), jnp.float32),
        ],
    )(level_weight, keys_weight)

    n_sc = _SC_ROWS
    n_tc = B - n_sc

    sc_kernel = _make_sc_kernel(n_sc, C, D, L)
    x_sc = jnp.pad(input[n_tc:], ((0, 0), (0, 32 - C))).reshape(n_sc * 32)
    out_sc = sc_kernel(
        x_sc,
        dlo.reshape(C * D),
        t.reshape(D),
        base.reshape(D),
    )

    BB, DB = 256, 512
    out_tc = pl.pallas_call(
        functools.partial(_tc_body, L=L),
        grid=(n_tc // BB, D // DB),
        in_specs=[
            pl.BlockSpec((BB, C), lambda i, j: (i, 0)),
            pl.BlockSpec((1, DB), lambda i, j: (0, j)),
            pl.BlockSpec((C, DB), lambda i, j: (0, j)),
            pl.BlockSpec((1, DB), lambda i, j: (0, j)),
        ],
        out_specs=pl.BlockSpec((BB, DB), lambda i, j: (i, j)),
        out_shape=jax.ShapeDtypeStruct((n_tc, D), jnp.float32),
    )(input[:n_tc], t, dlo, base)

    return jnp.concatenate([out_tc, out_sc.reshape(n_sc, D)], axis=0)


# speedup vs baseline: 3.2919x; 3.2919x over previous
import functools

import jax
import jax.numpy as jnp
from jax.experimental import pallas as pl
from jax.experimental.pallas import tpu as pltpu


def _pack_weights(D, W):
    d = jax.lax.broadcasted_iota(jnp.int32, (D, 2 * W), 0)
    col = jax.lax.broadcasted_iota(jnp.int32, (D, 2 * W), 1)
    word = d % W
    bit = d // W
    half = bit // 16
    hit = (col % W == word) & (col // W == half)
    val = (jnp.int32(1) << (bit % 16)).astype(jnp.float32)
    return jnp.where(hit, val, 0.0)


def _prep_body(lw_ref, keys_ref, x_ref, pb_ref, kb_ref, idx_ref, *, L, W):
    D = lw_ref.shape[1]
    m = _pack_weights(D, W).astype(jnp.bfloat16)
    lw_bits = (lw_ref[...] < 0).astype(jnp.bfloat16)
    pk = jnp.dot(lw_bits, m, preferred_element_type=jnp.float32)
    pb_ref[...] = (pk[:, :W].astype(jnp.int32)
                   | (pk[:, W:].astype(jnp.int32) << 16))
    k_bits = (keys_ref[...] < 0).astype(jnp.bfloat16)
    kk = jnp.dot(k_bits, m, preferred_element_type=jnp.float32)
    kb_ref[...] = (kk[:, :W].astype(jnp.int32)
                   | (kk[:, W:].astype(jnp.int32) << 16))
    x = x_ref[...]
    idx_ref[...] = jnp.clip(jnp.round(x * (L - 1)), 0.0, L - 1.0).astype(jnp.int32)


def _csa_popcount_lt13(words):
    pools = {0: list(words)}
    planes = {}
    w = 0
    while pools.get(w):
        pool = pools[w]
        while len(pool) >= 3:
            a, b, cn = pool.pop(), pool.pop(), pool.pop()
            t = a ^ b
            s = t ^ cn
            carry = (a & b) | (cn & t)
            pool.append(s)
            pools.setdefault(w + 1, []).append(carry)
        if len(pool) == 2:
            a, b = pool.pop(), pool.pop()
            pool.append(a ^ b)
            pools.setdefault(w + 1, []).append(a & b)
        planes[w] = pool[0]
        w += 1
    z = jnp.zeros_like(words[0])
    p = [planes.get(i, z) for i in range(5)]
    return ~p[4] & (~p[3] | ~p[2] | (~p[1] & ~p[0]))


def _main_body(idx_ref, pb_ref, kb_ref, out_ref, asm_ref, *, C, W):
    BB = out_ref.shape[0]
    for g in range(BB // 8):
        words = []
        for c in range(C):
            slot = c % 2
            for s in range(8):
                r = idx_ref[g * 8 + s, c]
                asm_ref[slot, s, :] = pb_ref[r, :]
            kb_c = jnp.broadcast_to(kb_ref[c:c + 1, :], (8, W))
            words.append(asm_ref[slot] ^ kb_c)
        pos = _csa_popcount_lt13(words)
        for m in range(32):
            bit = (pos << (31 - m)) < 0
            out_ref[g * 8:(g + 1) * 8, m * W:(m + 1) * W] = (
                jnp.where(bit, 1.0, -1.0))


@jax.jit
def kernel(input, keys_weight, level_weight):
    B, C = input.shape
    L, D = level_weight.shape
    W = 128
    pb, kb, idx = pl.pallas_call(
        functools.partial(_prep_body, L=L, W=W),
        grid=(1,),
        in_specs=[
            pl.BlockSpec((L, D), lambda i: (0, 0)),
            pl.BlockSpec((C, D), lambda i: (0, 0)),
            pl.BlockSpec((B, C), lambda i: (0, 0)),
        ],
        out_specs=[
            pl.BlockSpec((L, W), lambda i: (0, 0)),
            pl.BlockSpec((C, W), lambda i: (0, 0)),
            pl.BlockSpec((B, C), lambda i: (0, 0)),
        ],
        out_shape=[
            jax.ShapeDtypeStruct((L, W), jnp.int32),
            jax.ShapeDtypeStruct((C, W), jnp.int32),
            jax.ShapeDtypeStruct((B, C), jnp.int32),
        ],
    )(level_weight, keys_weight, input)

    BB = 64
    out = pl.pallas_call(
        functools.partial(_main_body, C=C, W=W),
        grid=(B // BB,),
        in_specs=[
            pl.BlockSpec((BB, C), lambda i: (i, 0), memory_space=pltpu.SMEM),
            pl.BlockSpec((L, W), lambda i: (0, 0)),
            pl.BlockSpec((C, W), lambda i: (0, 0)),
        ],
        out_specs=pl.BlockSpec((BB, D), lambda i: (i, 0)),
        out_shape=jax.ShapeDtypeStruct((B, D), jnp.float32),
        scratch_shapes=[pltpu.VMEM((2, 8, W), jnp.int32)],
    )(idx, pb, kb)
    return out
